# baseline (speedup 1.0000x reference)
import functools

import jax
import jax.numpy as jnp
from jax import lax
from jax.experimental import pallas as pl
from jax.experimental.pallas import tpu as pltpu
from jax.experimental.pallas import tpu_sc as plsc

NC = 2
NS = 16
NW = NC * NS
CS = 125
GC = 16
BN = 1000


def _sc_mesh():
    return plsc.VectorSubcoreMesh(
        core_axis_name="c", subcore_axis_name="s",
        num_cores=NC, num_subcores=NS)


def _make_deg(NP, N, epw):
    RPT = NP // NS

    def body(dst_hbm, out_hbm, dflat, hist, red_v, out_v, slab):
        c = lax.axis_index("c")
        s = lax.axis_index("s")
        wid = c * NS + s

        def zrow(i, carry):
            hist[pl.ds(i * 16, 16)] = jnp.zeros((16,), jnp.float32)
            return carry
        lax.fori_loop(0, NP // 16, zrow, 0)

        pltpu.sync_copy(dst_hbm.at[wid], dflat)
        ones = jnp.ones((16,), jnp.float32)

        def step(i, carry):
            idx = dflat[pl.ds(i * 16, 16)]
            plsc.addupdate_scatter(hist, [idx], ones)
            return carry
        lax.fori_loop(0, epw // 16, step, 0)

        pltpu.sync_copy(hist, slab.at[s, 0])
        plsc.subcore_barrier()
        pltpu.sync_copy(slab.at[:, :, pl.ds(s * RPT, RPT)], red_v)

        def red(l, carry):
            v = red_v[0, 0, pl.ds(l * 16, 16)]
            for r in range(1, NS):
                v = v + red_v[r, 0, pl.ds(l * 16, 16)]
            out_v[0, pl.ds(l * 16, 16)] = v
            return carry
        lax.fori_loop(0, RPT // 16, red, 0)
        pltpu.sync_copy(out_v, out_hbm.at[c, :, pl.ds(s * RPT, RPT)])

    return pl.kernel(
        body,
        out_type=jax.ShapeDtypeStruct((NC, 1, NP), jnp.float32),
        mesh=_sc_mesh(),
        scratch_types=[
            pltpu.VMEM((epw,), jnp.int32),
            pltpu.VMEM((NP,), jnp.float32),
            pltpu.VMEM((NS, 1, RPT), jnp.float32),
            pltpu.VMEM((1, RPT), jnp.float32),
            pltpu.VMEM_SHARED((NS, 1, NP), jnp.float32),
        ],
        compiler_params=pltpu.CompilerParams(needs_layout_passes=False),
    )


def _make_prop(NP, N, D, nchunk):
    RPT = NP // NS
    ZC = 80
    NZC = RPT // ZC
    NG = nchunk // GC
    H2 = GC // 2

    def body(u_hbm, src_hbm, dst_hbm, out_hbm,
             srcA, dstA, srcB, dstB, bufa, bufb, acc,
             sema, semb, semiA, semiB):
        c = lax.axis_index("c")
        s = lax.axis_index("s")
        wid = c * NS + s

        pltpu.async_copy(src_hbm.at[wid, pl.ds(0, GC)], srcA, semiA)
        pltpu.async_copy(dst_hbm.at[wid, pl.ds(0, GC)], dstA, semiA)

        def zrow(i, carry):
            for l in range(D // 16):
                bufa[i, pl.ds(l * 16, 16)] = jnp.zeros((16,), jnp.float32)
            return carry
        lax.fori_loop(0, ZC, zrow, 0)
        for r in range(NZC):
            pltpu.async_copy(bufa.at[pl.ds(0, ZC)],
                             acc.at[pl.ds(s * RPT + r * ZC, ZC)], semb)
        for r in range(NZC):
            pltpu.make_async_copy(bufa.at[pl.ds(0, ZC)],
                                  acc.at[pl.ds(s * RPT + r * ZC, ZC)],
                                  semb).wait()

        pltpu.make_async_copy(src_hbm.at[wid, pl.ds(0, GC)], srcA, semiA).wait()
        pltpu.make_async_copy(dst_hbm.at[wid, pl.ds(0, GC)], dstA, semiA).wait()
        plsc.subcore_barrier()

        pltpu.async_copy(u_hbm.at[srcA.at[0]], bufa, sema)

        for g in range(NG):
            S, Dv = (srcA, dstA) if g % 2 == 0 else (srcB, dstB)
            S2, D2 = (srcB, dstB) if g % 2 == 0 else (srcA, dstA)
            sem2 = semiB if g % 2 == 0 else semiA
            if g + 1 < NG:
                off = (g + 1) * GC
                pltpu.async_copy(src_hbm.at[wid, pl.ds(off, GC)], S2, sem2)
                pltpu.async_copy(dst_hbm.at[wid, pl.ds(off, GC)], D2, sem2)

            def step(j2, carry, S=S, Dv=Dv):
                j = j2 * 2
                pltpu.async_copy(u_hbm.at[S.at[j + 1]], bufb, semb)
                pltpu.make_async_copy(u_hbm.at[S.at[j]], bufa, sema).wait()
                pltpu.sync_copy(bufa, acc.at[Dv.at[j]], add=True)

                @pl.when(j2 < H2 - 1)
                def _():
                    pltpu.async_copy(u_hbm.at[S.at[j + 2]], bufa, sema)
                pltpu.make_async_copy(u_hbm.at[S.at[j + 1]], bufb, semb).wait()
                pltpu.sync_copy(bufb, acc.at[Dv.at[j + 1]], add=True)
                return carry
            lax.fori_loop(0, H2, step, 0)

            if g + 1 < NG:
                off = (g + 1) * GC
                pltpu.make_async_copy(src_hbm.at[wid, pl.ds(off, GC)], S2, sem2).wait()
                pltpu.make_async_copy(dst_hbm.at[wid, pl.ds(off, GC)], D2, sem2).wait()
                pltpu.async_copy(u_hbm.at[S2.at[0]], bufa, sema)

        plsc.subcore_barrier()
        pltpu.sync_copy(acc.at[pl.ds(s * RPT, RPT)],
                        out_hbm.at[c, pl.ds(s * RPT, RPT)])

    return pl.kernel(
        body,
        out_type=jax.ShapeDtypeStruct((NC, NP, D), jnp.float32),
        mesh=_sc_mesh(),
        scratch_types=[
            pltpu.VMEM((GC, CS), jnp.int32),
            pltpu.VMEM((GC, CS), jnp.int32),
            pltpu.VMEM((GC, CS), jnp.int32),
            pltpu.VMEM((GC, CS), jnp.int32),
            pltpu.VMEM((CS, D), jnp.float32),
            pltpu.VMEM((CS, D), jnp.float32),
            pltpu.VMEM_SHARED((NP, D), jnp.float32),
            pltpu.SemaphoreType.DMA,
            pltpu.SemaphoreType.DMA,
            pltpu.SemaphoreType.DMA,
            pltpu.SemaphoreType.DMA,
        ],
    )


def _dot(a, b):
    return jnp.dot(a, b, preferred_element_type=jnp.float32,
                   precision=lax.Precision.HIGHEST)


def _prep_body(x_ref, da_ref, dis_ref, u_ref):
    deg = da_ref[0] + da_ref[1]
    dis = jnp.where(deg > 0, lax.rsqrt(jnp.maximum(deg, 1.0)), 0.0)
    dis_ref[...] = dis
    u_ref[...] = x_ref[...] * dis


def _make_prep(N, Din, interpret=False):
    return pl.pallas_call(
        _prep_body,
        grid=(N // BN,),
        in_specs=[
            pl.BlockSpec((BN, Din), lambda i: (i, 0)),
            pl.BlockSpec((NC, BN, 1), lambda i: (0, i, 0)),
        ],
        out_specs=[
            pl.BlockSpec((BN, 1), lambda i: (i, 0)),
            pl.BlockSpec((BN, Din), lambda i: (i, 0)),
        ],
        out_shape=[
            jax.ShapeDtypeStruct((N, 1), jnp.float32),
            jax.ShapeDtypeStruct((N, Din), jnp.float32),
        ],
        interpret=interpret,
    )


def _scale_body(a_ref, dis_ref, u_ref):
    dis = dis_ref[...]
    u_ref[...] = (a_ref[0] + a_ref[1]) * (dis * dis)


def _make_scale(N, D, interpret=False):
    return pl.pallas_call(
        _scale_body,
        grid=(N // BN,),
        in_specs=[
            pl.BlockSpec((NC, BN, D), lambda i: (0, i, 0)),
            pl.BlockSpec((BN, 1), lambda i: (i, 0)),
        ],
        out_specs=pl.BlockSpec((BN, D), lambda i: (i, 0)),
        out_shape=jax.ShapeDtypeStruct((N, D), jnp.float32),
        interpret=interpret,
    )


def _step_body(a_ref, dis_ref, w_ref, oin_ref, u_ref, out_ref):
    dis = dis_ref[...]
    h = (a_ref[0] + a_ref[1]) * dis
    u_ref[...] = h * dis
    out_ref[...] = oin_ref[...] + _dot(h, w_ref[...])


def _make_step(N, D, Wd, interpret=False):
    return pl.pallas_call(
        _step_body,
        grid=(N // BN,),
        in_specs=[
            pl.BlockSpec((NC, BN, D), lambda i: (0, i, 0)),
            pl.BlockSpec((BN, 1), lambda i: (i, 0)),
            pl.BlockSpec((D, Wd), lambda i: (0, 0)),
            pl.BlockSpec((BN, Wd), lambda i: (i, 0)),
        ],
        out_specs=[
            pl.BlockSpec((BN, D), lambda i: (i, 0)),
            pl.BlockSpec((BN, Wd), lambda i: (i, 0)),
        ],
        out_shape=[
            jax.ShapeDtypeStruct((N, D), jnp.float32),
            jax.ShapeDtypeStruct((N, Wd), jnp.float32),
        ],
        interpret=interpret,
    )


def _mm_first_body(x_ref, w_ref, out_ref):
    out_ref[...] = _dot(x_ref[...], w_ref[...])


def _make_mm_first(N, D, Wd, interpret=False):
    return pl.pallas_call(
        _mm_first_body,
        grid=(N // BN,),
        in_specs=[
            pl.BlockSpec((BN, D), lambda i: (i, 0)),
            pl.BlockSpec((D, Wd), lambda i: (0, 0)),
        ],
        out_specs=pl.BlockSpec((BN, Wd), lambda i: (i, 0)),
        out_shape=jax.ShapeDtypeStruct((N, Wd), jnp.float32),
        interpret=interpret,
    )


def _mm_acc_body(a_ref, dis_ref, w_ref, oin_ref, out_ref):
    h = (a_ref[0] + a_ref[1]) * dis_ref[...]
    out_ref[...] = oin_ref[...] + _dot(h, w_ref[...])


def _make_mm_acc(N, D, Wd, interpret=False):
    return pl.pallas_call(
        _mm_acc_body,
        grid=(N // BN,),
        in_specs=[
            pl.BlockSpec((NC, BN, D), lambda i: (0, i, 0)),
            pl.BlockSpec((BN, 1), lambda i: (i, 0)),
            pl.BlockSpec((D, Wd), lambda i: (0, 0)),
            pl.BlockSpec((BN, Wd), lambda i: (i, 0)),
        ],
        out_specs=pl.BlockSpec((BN, Wd), lambda i: (i, 0)),
        out_shape=jax.ShapeDtypeStruct((N, Wd), jnp.float32),
        interpret=interpret,
    )


def _trans_body(a_ref, dis_ref, wk_ref, oin_ref, b_ref, hh_ref, u_ref):
    dis = dis_ref[...]
    h = (a_ref[0] + a_ref[1]) * dis
    o1 = oin_ref[...] + _dot(h, wk_ref[...]) + b_ref[...]
    hh = jnp.maximum(o1, 0.0)
    hh_ref[...] = hh
    u_ref[...] = hh * dis


def _make_trans(N, D, interpret=False):
    return pl.pallas_call(
        _trans_body,
        grid=(N // BN,),
        in_specs=[
            pl.BlockSpec((NC, BN, D), lambda i: (0, i, 0)),
            pl.BlockSpec((BN, 1), lambda i: (i, 0)),
            pl.BlockSpec((D, D), lambda i: (0, 0)),
            pl.BlockSpec((BN, D), lambda i: (i, 0)),
            pl.BlockSpec((1, D), lambda i: (0, 0)),
        ],
        out_specs=[
            pl.BlockSpec((BN, D), lambda i: (i, 0)),
            pl.BlockSpec((BN, D), lambda i: (i, 0)),
        ],
        out_shape=[
            jax.ShapeDtypeStruct((N, D), jnp.float32),
            jax.ShapeDtypeStruct((N, D), jnp.float32),
        ],
        interpret=interpret,
    )


def _final_body(a_ref, dis_ref, w_ref, oin_ref, b_ref, out_ref):
    sacc = a_ref[0] + a_ref[1]
    h = sacc * dis_ref[...]
    out_ref[...] = oin_ref[...] + _dot(h, w_ref[...]) + b_ref[...]


def _make_final(N, D, Dout, interpret=False):
    return pl.pallas_call(
        _final_body,
        grid=(N // BN,),
        in_specs=[
            pl.BlockSpec((NC, BN, D), lambda i: (0, i, 0)),
            pl.BlockSpec((BN, 1), lambda i: (i, 0)),
            pl.BlockSpec((D, Dout), lambda i: (0, 0)),
            pl.BlockSpec((BN, Dout), lambda i: (i, 0)),
            pl.BlockSpec((1, Dout), lambda i: (0, 0)),
        ],
        out_specs=pl.BlockSpec((BN, Dout), lambda i: (i, 0)),
        out_shape=jax.ShapeDtypeStruct((N, Dout), jnp.float32),
        interpret=interpret,
    )


def kernel(x, edge_index, W1, b1, W2, b2):
    N, Din = x.shape
    Kp1, _, H = W1.shape
    Dout = W2.shape[2]
    K = Kp1 - 1
    E = edge_index.shape[1]

    epw = -(-E // NW)
    nchunk = -(-epw // CS)
    nchunk = ((nchunk + GC - 1) // GC) * GC
    e_pad = NW * nchunk * CS
    NP = ((N + 8 + 2047) // 2048) * 2048

    src = edge_index[0]
    dst = edge_index[1]
    if e_pad != E:
        src = jnp.concatenate([src, jnp.zeros((e_pad - E,), src.dtype)])
        dst = jnp.concatenate([dst, jnp.full((e_pad - E,), N, dst.dtype)])
    src3 = src.reshape(NW, nchunk, CS)
    dst3 = dst.reshape(NW, nchunk, CS)
    dst2 = dst.reshape(NW, nchunk * CS)

    deg_fn = _make_deg(NP, N, nchunk * CS)
    prop_fn = _make_prop(NP, N, H, nchunk)
    scale_fn = _make_scale(N, H)
    mm_h = _make_mm_acc(N, H, H)
    mm_o = _make_mm_acc(N, H, Dout)

    dacc = deg_fn(dst2).reshape(NC, NP, 1)
    dis, u = _make_prep(N, Din)(x, dacc)
    out1 = _make_mm_first(N, Din, H)(x, W1[0])

    for k in range(1, K):
        acc = prop_fn(u, src3, dst3)
        u, out1 = _make_step(N, H, H)(acc, dis, W1[k], out1)

    acc = prop_fn(u, src3, dst3)
    hh, u = _make_trans(N, H)(acc, dis, W1[K], out1, b1.reshape(1, H))
    out2 = _make_mm_first(N, H, Dout)(hh, W2[0])

    for k in range(1, K):
        acc = prop_fn(u, src3, dst3)
        u, out2 = _make_step(N, H, Dout)(acc, dis, W2[k], out2)

    acc = prop_fn(u, src3, dst3)
    out = _make_final(N, H, Dout)(
        acc, dis, W2[K], out2, b2.reshape(1, Dout))
    return out


# speedup vs baseline: 1.0344x; 1.0344x over previous
import functools

import jax
import jax.numpy as jnp
from jax import lax
from jax.experimental import pallas as pl
from jax.experimental.pallas import tpu as pltpu
from jax.experimental.pallas import tpu_sc as plsc

NC = 2
NS = 16
NW = NC * NS
CS = 125
GC = 16
BN = 2000


def _sc_mesh():
    return plsc.VectorSubcoreMesh(
        core_axis_name="c", subcore_axis_name="s",
        num_cores=NC, num_subcores=NS)


def _make_deg(NP, N, epw):
    RPT = NP // NS

    def body(dst_hbm, out_hbm, dflat, hist, red_v, out_v, slab):
        c = lax.axis_index("c")
        s = lax.axis_index("s")
        wid = c * NS + s

        def zrow(i, carry):
            hist[pl.ds(i * 16, 16)] = jnp.zeros((16,), jnp.float32)
            return carry
        lax.fori_loop(0, NP // 16, zrow, 0)

        pltpu.sync_copy(dst_hbm.at[wid], dflat)
        ones = jnp.ones((16,), jnp.float32)

        def step(i, carry):
            idx = dflat[pl.ds(i * 16, 16)]
            plsc.addupdate_scatter(hist, [idx], ones)
            return carry
        lax.fori_loop(0, epw // 16, step, 0)

        pltpu.sync_copy(hist, slab.at[s, 0])
        plsc.subcore_barrier()
        pltpu.sync_copy(slab.at[:, :, pl.ds(s * RPT, RPT)], red_v)

        def red(l, carry):
            v = red_v[0, 0, pl.ds(l * 16, 16)]
            for r in range(1, NS):
                v = v + red_v[r, 0, pl.ds(l * 16, 16)]
            out_v[0, pl.ds(l * 16, 16)] = v
            return carry
        lax.fori_loop(0, RPT // 16, red, 0)
        pltpu.sync_copy(out_v, out_hbm.at[c, :, pl.ds(s * RPT, RPT)])

    return pl.kernel(
        body,
        out_type=jax.ShapeDtypeStruct((NC, 1, NP), jnp.float32),
        mesh=_sc_mesh(),
        scratch_types=[
            pltpu.VMEM((epw,), jnp.int32),
            pltpu.VMEM((NP,), jnp.float32),
            pltpu.VMEM((NS, 1, RPT), jnp.float32),
            pltpu.VMEM((1, RPT), jnp.float32),
            pltpu.VMEM_SHARED((NS, 1, NP), jnp.float32),
        ],
        compiler_params=pltpu.CompilerParams(needs_layout_passes=False),
    )


def _make_prop(NP, N, D, nchunk):
    RPT = NP // NS
    ZC = 80
    NZC = RPT // ZC
    NG = nchunk // GC
    H2 = GC // 2

    def body(u_hbm, src_hbm, dst_hbm, out_hbm,
             srcA, dstA, srcB, dstB, bufa, bufb, acc,
             sema, semb, semiA, semiB):
        c = lax.axis_index("c")
        s = lax.axis_index("s")
        wid = c * NS + s

        pltpu.async_copy(src_hbm.at[wid, pl.ds(0, GC)], srcA, semiA)
        pltpu.async_copy(dst_hbm.at[wid, pl.ds(0, GC)], dstA, semiA)

        def zrow(i, carry):
            for l in range(D // 16):
                bufa[i, pl.ds(l * 16, 16)] = jnp.zeros((16,), jnp.float32)
            return carry
        lax.fori_loop(0, ZC, zrow, 0)
        for r in range(NZC):
            pltpu.async_copy(bufa.at[pl.ds(0, ZC)],
                             acc.at[pl.ds(s * RPT + r * ZC, ZC)], semb)
        for r in range(NZC):
            pltpu.make_async_copy(bufa.at[pl.ds(0, ZC)],
                                  acc.at[pl.ds(s * RPT + r * ZC, ZC)],
                                  semb).wait()

        pltpu.make_async_copy(src_hbm.at[wid, pl.ds(0, GC)], srcA, semiA).wait()
        pltpu.make_async_copy(dst_hbm.at[wid, pl.ds(0, GC)], dstA, semiA).wait()
        plsc.subcore_barrier()

        pltpu.async_copy(u_hbm.at[srcA.at[0]], bufa, sema)

        for g in range(NG):
            S, Dv = (srcA, dstA) if g % 2 == 0 else (srcB, dstB)
            S2, D2 = (srcB, dstB) if g % 2 == 0 else (srcA, dstA)
            sem2 = semiB if g % 2 == 0 else semiA
            if g + 1 < NG:
                off = (g + 1) * GC
                pltpu.async_copy(src_hbm.at[wid, pl.ds(off, GC)], S2, sem2)
                pltpu.async_copy(dst_hbm.at[wid, pl.ds(off, GC)], D2, sem2)

            def step(j2, carry, S=S, Dv=Dv):
                j = j2 * 2
                pltpu.async_copy(u_hbm.at[S.at[j + 1]], bufb, semb)
                pltpu.make_async_copy(u_hbm.at[S.at[j]], bufa, sema).wait()
                pltpu.sync_copy(bufa, acc.at[Dv.at[j]], add=True)

                @pl.when(j2 < H2 - 1)
                def _():
                    pltpu.async_copy(u_hbm.at[S.at[j + 2]], bufa, sema)
                pltpu.make_async_copy(u_hbm.at[S.at[j + 1]], bufb, semb).wait()
                pltpu.sync_copy(bufb, acc.at[Dv.at[j + 1]], add=True)
                return carry
            lax.fori_loop(0, H2, step, 0)

            if g + 1 < NG:
                off = (g + 1) * GC
                pltpu.make_async_copy(src_hbm.at[wid, pl.ds(off, GC)], S2, sem2).wait()
                pltpu.make_async_copy(dst_hbm.at[wid, pl.ds(off, GC)], D2, sem2).wait()
                pltpu.async_copy(u_hbm.at[S2.at[0]], bufa, sema)

        plsc.subcore_barrier()
        pltpu.sync_copy(acc.at[pl.ds(s * RPT, RPT)],
                        out_hbm.at[c, pl.ds(s * RPT, RPT)])

    return pl.kernel(
        body,
        out_type=jax.ShapeDtypeStruct((NC, NP, D), jnp.float32),
        mesh=_sc_mesh(),
        scratch_types=[
            pltpu.VMEM((GC, CS), jnp.int32),
            pltpu.VMEM((GC, CS), jnp.int32),
            pltpu.VMEM((GC, CS), jnp.int32),
            pltpu.VMEM((GC, CS), jnp.int32),
            pltpu.VMEM((CS, D), jnp.float32),
            pltpu.VMEM((CS, D), jnp.float32),
            pltpu.VMEM_SHARED((NP, D), jnp.float32),
            pltpu.SemaphoreType.DMA,
            pltpu.SemaphoreType.DMA,
            pltpu.SemaphoreType.DMA,
            pltpu.SemaphoreType.DMA,
        ],
    )


def _dot(a, b):
    return jnp.dot(a, b, preferred_element_type=jnp.float32,
                   precision=lax.Precision.HIGHEST)


def _prep_body(x_ref, da_ref, dis_ref, u_ref):
    deg = da_ref[0] + da_ref[1]
    dis = jnp.where(deg > 0, lax.rsqrt(jnp.maximum(deg, 1.0)), 0.0)
    dis_ref[...] = dis
    u_ref[...] = x_ref[...] * dis


def _make_prep(N, Din, interpret=False):
    return pl.pallas_call(
        _prep_body,
        grid=(N // BN,),
        in_specs=[
            pl.BlockSpec((BN, Din), lambda i: (i, 0)),
            pl.BlockSpec((NC, BN, 1), lambda i: (0, i, 0)),
        ],
        out_specs=[
            pl.BlockSpec((BN, 1), lambda i: (i, 0)),
            pl.BlockSpec((BN, Din), lambda i: (i, 0)),
        ],
        out_shape=[
            jax.ShapeDtypeStruct((N, 1), jnp.float32),
            jax.ShapeDtypeStruct((N, Din), jnp.float32),
        ],
        interpret=interpret,
    )


def _scale_body(a_ref, dis_ref, u_ref):
    dis = dis_ref[...]
    u_ref[...] = (a_ref[0] + a_ref[1]) * (dis * dis)


def _make_scale(N, D, interpret=False):
    return pl.pallas_call(
        _scale_body,
        grid=(N // BN,),
        in_specs=[
            pl.BlockSpec((NC, BN, D), lambda i: (0, i, 0)),
            pl.BlockSpec((BN, 1), lambda i: (i, 0)),
        ],
        out_specs=pl.BlockSpec((BN, D), lambda i: (i, 0)),
        out_shape=jax.ShapeDtypeStruct((N, D), jnp.float32),
        interpret=interpret,
    )


def _mm_first_body(x_ref, w_ref, out_ref):
    out_ref[...] = _dot(x_ref[...], w_ref[...])


def _make_mm_first(N, D, Wd, interpret=False):
    return pl.pallas_call(
        _mm_first_body,
        grid=(N // BN,),
        in_specs=[
            pl.BlockSpec((BN, D), lambda i: (i, 0)),
            pl.BlockSpec((D, Wd), lambda i: (0, 0)),
        ],
        out_specs=pl.BlockSpec((BN, Wd), lambda i: (i, 0)),
        out_shape=jax.ShapeDtypeStruct((N, Wd), jnp.float32),
        interpret=interpret,
    )


def _mm_acc_body(a_ref, dis_ref, w_ref, oin_ref, out_ref):
    h = (a_ref[0] + a_ref[1]) * dis_ref[...]
    out_ref[...] = oin_ref[...] + _dot(h, w_ref[...])


def _make_mm_acc(N, D, Wd, interpret=False):
    return pl.pallas_call(
        _mm_acc_body,
        grid=(N // BN,),
        in_specs=[
            pl.BlockSpec((NC, BN, D), lambda i: (0, i, 0)),
            pl.BlockSpec((BN, 1), lambda i: (i, 0)),
            pl.BlockSpec((D, Wd), lambda i: (0, 0)),
            pl.BlockSpec((BN, Wd), lambda i: (i, 0)),
        ],
        out_specs=pl.BlockSpec((BN, Wd), lambda i: (i, 0)),
        out_shape=jax.ShapeDtypeStruct((N, Wd), jnp.float32),
        interpret=interpret,
    )


def _trans_body(a_ref, dis_ref, wk_ref, oin_ref, b_ref, hh_ref, u_ref):
    dis = dis_ref[...]
    h = (a_ref[0] + a_ref[1]) * dis
    o1 = oin_ref[...] + _dot(h, wk_ref[...]) + b_ref[...]
    hh = jnp.maximum(o1, 0.0)
    hh_ref[...] = hh
    u_ref[...] = hh * dis


def _make_trans(N, D, interpret=False):
    return pl.pallas_call(
        _trans_body,
        grid=(N // BN,),
        in_specs=[
            pl.BlockSpec((NC, BN, D), lambda i: (0, i, 0)),
            pl.BlockSpec((BN, 1), lambda i: (i, 0)),
            pl.BlockSpec((D, D), lambda i: (0, 0)),
            pl.BlockSpec((BN, D), lambda i: (i, 0)),
            pl.BlockSpec((1, D), lambda i: (0, 0)),
        ],
        out_specs=[
            pl.BlockSpec((BN, D), lambda i: (i, 0)),
            pl.BlockSpec((BN, D), lambda i: (i, 0)),
        ],
        out_shape=[
            jax.ShapeDtypeStruct((N, D), jnp.float32),
            jax.ShapeDtypeStruct((N, D), jnp.float32),
        ],
        interpret=interpret,
    )


def _final_body(a_ref, dis_ref, w_ref, oin_ref, b_ref, out_ref):
    sacc = a_ref[0] + a_ref[1]
    h = sacc * dis_ref[...]
    out_ref[...] = oin_ref[...] + _dot(h, w_ref[...]) + b_ref[...]


def _make_final(N, D, Dout, interpret=False):
    return pl.pallas_call(
        _final_body,
        grid=(N // BN,),
        in_specs=[
            pl.BlockSpec((NC, BN, D), lambda i: (0, i, 0)),
            pl.BlockSpec((BN, 1), lambda i: (i, 0)),
            pl.BlockSpec((D, Dout), lambda i: (0, 0)),
            pl.BlockSpec((BN, Dout), lambda i: (i, 0)),
            pl.BlockSpec((1, Dout), lambda i: (0, 0)),
        ],
        out_specs=pl.BlockSpec((BN, Dout), lambda i: (i, 0)),
        out_shape=jax.ShapeDtypeStruct((N, Dout), jnp.float32),
        interpret=interpret,
    )


def kernel(x, edge_index, W1, b1, W2, b2):
    N, Din = x.shape
    Kp1, _, H = W1.shape
    Dout = W2.shape[2]
    K = Kp1 - 1
    E = edge_index.shape[1]

    epw = -(-E // NW)
    nchunk = -(-epw // CS)
    nchunk = ((nchunk + GC - 1) // GC) * GC
    e_pad = NW * nchunk * CS
    NP = ((N + 8 + 2047) // 2048) * 2048

    src = edge_index[0]
    dst = edge_index[1]
    if e_pad != E:
        src = jnp.concatenate([src, jnp.zeros((e_pad - E,), src.dtype)])
        dst = jnp.concatenate([dst, jnp.full((e_pad - E,), N, dst.dtype)])
    src3 = src.reshape(NW, nchunk, CS)
    dst3 = dst.reshape(NW, nchunk, CS)
    dst2 = dst.reshape(NW, nchunk * CS)

    deg_fn = _make_deg(NP, N, nchunk * CS)
    prop_fn = _make_prop(NP, N, H, nchunk)
    scale_fn = _make_scale(N, H)
    mm_h = _make_mm_acc(N, H, H)
    mm_o = _make_mm_acc(N, H, Dout)

    dacc = deg_fn(dst2).reshape(NC, NP, 1)
    dis, u = _make_prep(N, Din)(x, dacc)
    out1 = _make_mm_first(N, Din, H)(x, W1[0])

    for k in range(1, K):
        acc = prop_fn(u, src3, dst3)
        u = scale_fn(acc, dis)
        out1 = mm_h(acc, dis, W1[k], out1)

    acc = prop_fn(u, src3, dst3)
    hh, u = _make_trans(N, H)(acc, dis, W1[K], out1, b1.reshape(1, H))
    out2 = _make_mm_first(N, H, Dout)(hh, W2[0])

    for k in range(1, K):
        acc = prop_fn(u, src3, dst3)
        u = scale_fn(acc, dis)
        out2 = mm_o(acc, dis, W2[k], out2)

    acc = prop_fn(u, src3, dst3)
    out = _make_final(N, H, Dout)(
        acc, dis, W2[K], out2, b2.reshape(1, Dout))
    return out


# speedup vs baseline: 1.0355x; 1.0011x over previous
import functools

import jax
import jax.numpy as jnp
from jax import lax
from jax.experimental import pallas as pl
from jax.experimental.pallas import tpu as pltpu
from jax.experimental.pallas import tpu_sc as plsc

NC = 2
NS = 16
NW = NC * NS
CS = 125
GC = 16
BN = 5000


def _sc_mesh():
    return plsc.VectorSubcoreMesh(
        core_axis_name="c", subcore_axis_name="s",
        num_cores=NC, num_subcores=NS)


def _make_deg(NP, N, epw):
    RPT = NP // NS

    def body(dst_hbm, out_hbm, dflat, hist, red_v, out_v, slab):
        c = lax.axis_index("c")
        s = lax.axis_index("s")
        wid = c * NS + s

        def zrow(i, carry):
            hist[pl.ds(i * 16, 16)] = jnp.zeros((16,), jnp.float32)
            return carry
        lax.fori_loop(0, NP // 16, zrow, 0)

        pltpu.sync_copy(dst_hbm.at[wid], dflat)
        ones = jnp.ones((16,), jnp.float32)

        def step(i, carry):
            idx = dflat[pl.ds(i * 16, 16)]
            plsc.addupdate_scatter(hist, [idx], ones)
            return carry
        lax.fori_loop(0, epw // 16, step, 0)

        pltpu.sync_copy(hist, slab.at[s, 0])
        plsc.subcore_barrier()
        pltpu.sync_copy(slab.at[:, :, pl.ds(s * RPT, RPT)], red_v)

        def red(l, carry):
            v = red_v[0, 0, pl.ds(l * 16, 16)]
            for r in range(1, NS):
                v = v + red_v[r, 0, pl.ds(l * 16, 16)]
            out_v[0, pl.ds(l * 16, 16)] = v
            return carry
        lax.fori_loop(0, RPT // 16, red, 0)
        pltpu.sync_copy(out_v, out_hbm.at[c, :, pl.ds(s * RPT, RPT)])

    return pl.kernel(
        body,
        out_type=jax.ShapeDtypeStruct((NC, 1, NP), jnp.float32),
        mesh=_sc_mesh(),
        scratch_types=[
            pltpu.VMEM((epw,), jnp.int32),
            pltpu.VMEM((NP,), jnp.float32),
            pltpu.VMEM((NS, 1, RPT), jnp.float32),
            pltpu.VMEM((1, RPT), jnp.float32),
            pltpu.VMEM_SHARED((NS, 1, NP), jnp.float32),
        ],
        compiler_params=pltpu.CompilerParams(needs_layout_passes=False),
    )


def _make_prop(NP, N, D, nchunk):
    RPT = NP // NS
    ZC = 80
    NZC = RPT // ZC
    NG = nchunk // GC
    H2 = GC // 2

    def body(u_hbm, src_hbm, dst_hbm, out_hbm,
             srcA, dstA, srcB, dstB, bufa, bufb, acc,
             sema, semb, semiA, semiB):
        c = lax.axis_index("c")
        s = lax.axis_index("s")
        wid = c * NS + s

        pltpu.async_copy(src_hbm.at[wid, pl.ds(0, GC)], srcA, semiA)
        pltpu.async_copy(dst_hbm.at[wid, pl.ds(0, GC)], dstA, semiA)

        def zrow(i, carry):
            for l in range(D // 16):
                bufa[i, pl.ds(l * 16, 16)] = jnp.zeros((16,), jnp.float32)
            return carry
        lax.fori_loop(0, ZC, zrow, 0)
        for r in range(NZC):
            pltpu.async_copy(bufa.at[pl.ds(0, ZC)],
                             acc.at[pl.ds(s * RPT + r * ZC, ZC)], semb)
        for r in range(NZC):
            pltpu.make_async_copy(bufa.at[pl.ds(0, ZC)],
                                  acc.at[pl.ds(s * RPT + r * ZC, ZC)],
                                  semb).wait()

        pltpu.make_async_copy(src_hbm.at[wid, pl.ds(0, GC)], srcA, semiA).wait()
        pltpu.make_async_copy(dst_hbm.at[wid, pl.ds(0, GC)], dstA, semiA).wait()
        plsc.subcore_barrier()

        pltpu.async_copy(u_hbm.at[srcA.at[0]], bufa, sema)

        for g in range(NG):
            S, Dv = (srcA, dstA) if g % 2 == 0 else (srcB, dstB)
            S2, D2 = (srcB, dstB) if g % 2 == 0 else (srcA, dstA)
            sem2 = semiB if g % 2 == 0 else semiA
            if g + 1 < NG:
                off = (g + 1) * GC
                pltpu.async_copy(src_hbm.at[wid, pl.ds(off, GC)], S2, sem2)
                pltpu.async_copy(dst_hbm.at[wid, pl.ds(off, GC)], D2, sem2)

            def step(j2, carry, S=S, Dv=Dv):
                j = j2 * 2
                pltpu.async_copy(u_hbm.at[S.at[j + 1]], bufb, semb)
                pltpu.make_async_copy(u_hbm.at[S.at[j]], bufa, sema).wait()
                pltpu.sync_copy(bufa, acc.at[Dv.at[j]], add=True)

                @pl.when(j2 < H2 - 1)
                def _():
                    pltpu.async_copy(u_hbm.at[S.at[j + 2]], bufa, sema)
                pltpu.make_async_copy(u_hbm.at[S.at[j + 1]], bufb, semb).wait()
                pltpu.sync_copy(bufb, acc.at[Dv.at[j + 1]], add=True)
                return carry
            lax.fori_loop(0, H2, step, 0)

            if g + 1 < NG:
                off = (g + 1) * GC
                pltpu.make_async_copy(src_hbm.at[wid, pl.ds(off, GC)], S2, sem2).wait()
                pltpu.make_async_copy(dst_hbm.at[wid, pl.ds(off, GC)], D2, sem2).wait()
                pltpu.async_copy(u_hbm.at[S2.at[0]], bufa, sema)

        plsc.subcore_barrier()
        pltpu.sync_copy(acc.at[pl.ds(s * RPT, RPT)],
                        out_hbm.at[c, pl.ds(s * RPT, RPT)])

    return pl.kernel(
        body,
        out_type=jax.ShapeDtypeStruct((NC, NP, D), jnp.float32),
        mesh=_sc_mesh(),
        scratch_types=[
            pltpu.VMEM((GC, CS), jnp.int32),
            pltpu.VMEM((GC, CS), jnp.int32),
            pltpu.VMEM((GC, CS), jnp.int32),
            pltpu.VMEM((GC, CS), jnp.int32),
            pltpu.VMEM((CS, D), jnp.float32),
            pltpu.VMEM((CS, D), jnp.float32),
            pltpu.VMEM_SHARED((NP, D), jnp.float32),
            pltpu.SemaphoreType.DMA,
            pltpu.SemaphoreType.DMA,
            pltpu.SemaphoreType.DMA,
            pltpu.SemaphoreType.DMA,
        ],
    )


def _dot(a, b):
    return jnp.dot(a, b, preferred_element_type=jnp.float32,
                   precision=lax.Precision.HIGHEST)


def _prep_body(x_ref, da_ref, dis_ref, u_ref):
    deg = da_ref[0] + da_ref[1]
    dis = jnp.where(deg > 0, lax.rsqrt(jnp.maximum(deg, 1.0)), 0.0)
    dis_ref[...] = dis
    u_ref[...] = x_ref[...] * dis


def _make_prep(N, Din, interpret=False):
    return pl.pallas_call(
        _prep_body,
        grid=(N // BN,),
        in_specs=[
            pl.BlockSpec((BN, Din), lambda i: (i, 0)),
            pl.BlockSpec((NC, BN, 1), lambda i: (0, i, 0)),
        ],
        out_specs=[
            pl.BlockSpec((BN, 1), lambda i: (i, 0)),
            pl.BlockSpec((BN, Din), lambda i: (i, 0)),
        ],
        out_shape=[
            jax.ShapeDtypeStruct((N, 1), jnp.float32),
            jax.ShapeDtypeStruct((N, Din), jnp.float32),
        ],
        interpret=interpret,
    )


def _scale_body(a_ref, dis_ref, u_ref):
    dis = dis_ref[...]
    u_ref[...] = (a_ref[0] + a_ref[1]) * (dis * dis)


def _make_scale(N, D, interpret=False):
    return pl.pallas_call(
        _scale_body,
        grid=(N // BN,),
        in_specs=[
            pl.BlockSpec((NC, BN, D), lambda i: (0, i, 0)),
            pl.BlockSpec((BN, 1), lambda i: (i, 0)),
        ],
        out_specs=pl.BlockSpec((BN, D), lambda i: (i, 0)),
        out_shape=jax.ShapeDtypeStruct((N, D), jnp.float32),
        interpret=interpret,
    )


def _mm_first_body(x_ref, w_ref, out_ref):
    out_ref[...] = _dot(x_ref[...], w_ref[...])


def _make_mm_first(N, D, Wd, interpret=False):
    return pl.pallas_call(
        _mm_first_body,
        grid=(N // BN,),
        in_specs=[
            pl.BlockSpec((BN, D), lambda i: (i, 0)),
            pl.BlockSpec((D, Wd), lambda i: (0, 0)),
        ],
        out_specs=pl.BlockSpec((BN, Wd), lambda i: (i, 0)),
        out_shape=jax.ShapeDtypeStruct((N, Wd), jnp.float32),
        interpret=interpret,
    )


def _mm_acc_body(a_ref, dis_ref, w_ref, oin_ref, out_ref):
    h = (a_ref[0] + a_ref[1]) * dis_ref[...]
    out_ref[...] = oin_ref[...] + _dot(h, w_ref[...])


def _make_mm_acc(N, D, Wd, interpret=False):
    return pl.pallas_call(
        _mm_acc_body,
        grid=(N // BN,),
        in_specs=[
            pl.BlockSpec((NC, BN, D), lambda i: (0, i, 0)),
            pl.BlockSpec((BN, 1), lambda i: (i, 0)),
            pl.BlockSpec((D, Wd), lambda i: (0, 0)),
            pl.BlockSpec((BN, Wd), lambda i: (i, 0)),
        ],
        out_specs=pl.BlockSpec((BN, Wd), lambda i: (i, 0)),
        out_shape=jax.ShapeDtypeStruct((N, Wd), jnp.float32),
        interpret=interpret,
    )


def _trans_body(a_ref, dis_ref, wk_ref, oin_ref, b_ref, hh_ref, u_ref):
    dis = dis_ref[...]
    h = (a_ref[0] + a_ref[1]) * dis
    o1 = oin_ref[...] + _dot(h, wk_ref[...]) + b_ref[...]
    hh = jnp.maximum(o1, 0.0)
    hh_ref[...] = hh
    u_ref[...] = hh * dis


def _make_trans(N, D, interpret=False):
    return pl.pallas_call(
        _trans_body,
        grid=(N // BN,),
        in_specs=[
            pl.BlockSpec((NC, BN, D), lambda i: (0, i, 0)),
            pl.BlockSpec((BN, 1), lambda i: (i, 0)),
            pl.BlockSpec((D, D), lambda i: (0, 0)),
            pl.BlockSpec((BN, D), lambda i: (i, 0)),
            pl.BlockSpec((1, D), lambda i: (0, 0)),
        ],
        out_specs=[
            pl.BlockSpec((BN, D), lambda i: (i, 0)),
            pl.BlockSpec((BN, D), lambda i: (i, 0)),
        ],
        out_shape=[
            jax.ShapeDtypeStruct((N, D), jnp.float32),
            jax.ShapeDtypeStruct((N, D), jnp.float32),
        ],
        interpret=interpret,
    )


def _final_body(a_ref, dis_ref, w_ref, oin_ref, b_ref, out_ref):
    sacc = a_ref[0] + a_ref[1]
    h = sacc * dis_ref[...]
    out_ref[...] = oin_ref[...] + _dot(h, w_ref[...]) + b_ref[...]


def _make_final(N, D, Dout, interpret=False):
    return pl.pallas_call(
        _final_body,
        grid=(N // BN,),
        in_specs=[
            pl.BlockSpec((NC, BN, D), lambda i: (0, i, 0)),
            pl.BlockSpec((BN, 1), lambda i: (i, 0)),
            pl.BlockSpec((D, Dout), lambda i: (0, 0)),
            pl.BlockSpec((BN, Dout), lambda i: (i, 0)),
            pl.BlockSpec((1, Dout), lambda i: (0, 0)),
        ],
        out_specs=pl.BlockSpec((BN, Dout), lambda i: (i, 0)),
        out_shape=jax.ShapeDtypeStruct((N, Dout), jnp.float32),
        interpret=interpret,
    )


def kernel(x, edge_index, W1, b1, W2, b2):
    N, Din = x.shape
    Kp1, _, H = W1.shape
    Dout = W2.shape[2]
    K = Kp1 - 1
    E = edge_index.shape[1]

    epw = -(-E // NW)
    nchunk = -(-epw // CS)
    nchunk = ((nchunk + GC - 1) // GC) * GC
    e_pad = NW * nchunk * CS
    NP = ((N + 8 + 2047) // 2048) * 2048

    src = edge_index[0]
    dst = edge_index[1]
    if e_pad != E:
        src = jnp.concatenate([src, jnp.zeros((e_pad - E,), src.dtype)])
        dst = jnp.concatenate([dst, jnp.full((e_pad - E,), N, dst.dtype)])
    src3 = src.reshape(NW, nchunk, CS)
    dst3 = dst.reshape(NW, nchunk, CS)
    dst2 = dst.reshape(NW, nchunk * CS)

    deg_fn = _make_deg(NP, N, nchunk * CS)
    prop_fn = _make_prop(NP, N, H, nchunk)
    scale_fn = _make_scale(N, H)
    mm_h = _make_mm_acc(N, H, H)
    mm_o = _make_mm_acc(N, H, Dout)

    dacc = deg_fn(dst2).reshape(NC, NP, 1)
    dis, u = _make_prep(N, Din)(x, dacc)
    out1 = _make_mm_first(N, Din, H)(x, W1[0])

    for k in range(1, K):
        acc = prop_fn(u, src3, dst3)
        u = scale_fn(acc, dis)
        out1 = mm_h(acc, dis, W1[k], out1)

    acc = prop_fn(u, src3, dst3)
    hh, u = _make_trans(N, H)(acc, dis, W1[K], out1, b1.reshape(1, H))
    out2 = _make_mm_first(N, H, Dout)(hh, W2[0])

    for k in range(1, K):
        acc = prop_fn(u, src3, dst3)
        u = scale_fn(acc, dis)
        out2 = mm_o(acc, dis, W2[k], out2)

    acc = prop_fn(u, src3, dst3)
    out = _make_final(N, H, Dout)(
        acc, dis, W2[K], out2, b2.reshape(1, Dout))
    return out
